# baseline (speedup 1.0000x reference)
import jax
import jax.numpy as jnp
from jax.experimental import pallas as pl

N = 20000
D = 3
KNN = 17
LANES = 128
G = (N + LANES - 1) // LANES
NPAD = G * LANES
QB = 8
R = 6
BIGI = 2**30


def _knn_body(xq_ref, k0_ref, k1_ref, k2_ref, out_ref):
    s = pl.program_id(0)
    xq = xq_ref[...]
    rows = s * QB + jax.lax.broadcasted_iota(jnp.int32, (QB, 1), 0)
    q2 = jnp.sum(xq * xq, axis=1, keepdims=True)
    xqb = xq.astype(jnp.bfloat16).astype(jnp.float32)
    qc = [xqb[:, c:c + 1] for c in range(D)]

    lane = jax.lax.broadcasted_iota(jnp.int32, (QB, LANES), 1)
    inf = jnp.full((QB, LANES), jnp.inf, dtype=jnp.float32)

    def body(g, carry):
        lv, li = carry
        base = g * LANES
        kt = [k0_ref[pl.ds(g, 1), :], k1_ref[pl.ds(g, 1), :],
              k2_ref[pl.ds(g, 1), :]]
        k2 = kt[0] * kt[0] + kt[1] * kt[1] + kt[2] * kt[2]
        kb = [k.astype(jnp.bfloat16).astype(jnp.float32) for k in kt]
        dot = qc[0] * kb[0] + qc[1] * kb[1] + qc[2] * kb[2]
        d2 = (q2 + k2) - 2.0 * dot
        ki = base + lane
        valid = (ki != rows) & (ki < N)
        v = jnp.where(valid, d2, inf)
        vi = ki
        lv2, li2 = [], []
        for r in range(R):
            cmp = v < lv[r]
            lv2.append(jnp.where(cmp, v, lv[r]))
            li2.append(jnp.where(cmp, vi, li[r]))
            v = jnp.where(cmp, lv[r], v)
            vi = jnp.where(cmp, li[r], vi)
        return lv2, li2

    init_lv = [inf for _ in range(R)]
    init_li = [jnp.zeros((QB, LANES), jnp.int32) for _ in range(R)]
    lv, li = jax.lax.fori_loop(0, G, body, (init_lv, init_li))

    pv = jnp.concatenate(lv, axis=1)
    pi = jnp.concatenate(li, axis=1)

    mprev = jnp.full((QB, 1), -jnp.inf, dtype=jnp.float32)
    iprev = jnp.full((QB, 1), -1, dtype=jnp.int32)
    col = jax.lax.broadcasted_iota(jnp.int32, (QB, LANES), 1)
    res = jnp.zeros((QB, LANES), jnp.int32)
    pinf = jnp.full(pv.shape, jnp.inf, dtype=jnp.float32)
    pbig = jnp.full(pv.shape, BIGI, dtype=jnp.int32)
    for j in range(KNN):
        active = (pv > mprev) | ((pv == mprev) & (pi > iprev))
        vals = jnp.where(active, pv, pinf)
        m = jnp.min(vals, axis=1, keepdims=True)
        idx = jnp.min(jnp.where(vals == m, pi, pbig), axis=1, keepdims=True)
        res = jnp.where(col == j, idx, res)
        mprev, iprev = m, idx
    out_ref[...] = res


def kernel(datapoint):
    x = datapoint.astype(jnp.float32)
    xpad = jnp.pad(x, ((0, NPAD - N), (0, 0)))
    kcoord = [xpad[:, c].reshape(G, LANES) for c in range(D)]
    nbr = pl.pallas_call(
        _knn_body,
        grid=(NPAD // QB,),
        in_specs=[
            pl.BlockSpec((QB, D), lambda s: (s, 0)),
            pl.BlockSpec((G, LANES), lambda s: (0, 0)),
            pl.BlockSpec((G, LANES), lambda s: (0, 0)),
            pl.BlockSpec((G, LANES), lambda s: (0, 0)),
        ],
        out_specs=pl.BlockSpec((QB, LANES), lambda s: (s, 0)),
        out_shape=jax.ShapeDtypeStruct((NPAD, LANES), jnp.int32),
    )(xpad, *kcoord)
    src = nbr[:N, :KNN].reshape(-1)
    dst = jnp.repeat(jnp.arange(N), KNN)
    return jnp.stack([src, dst], axis=0).astype(jnp.int64)


# speedup vs baseline: 3.2739x; 3.2739x over previous
import jax
import jax.numpy as jnp
from jax.experimental import pallas as pl

N = 20000
D = 3
KNN = 17
LANES = 128
G = (N + LANES - 1) // LANES
NPAD = G * LANES
QB = 8
R = 6
BIGI = 2**30


def _knn_body(xq_ref, k0_ref, k1_ref, k2_ref, out_ref):
    s = pl.program_id(0)
    xq = xq_ref[...]
    rows = s * QB + jax.lax.broadcasted_iota(jnp.int32, (QB, 1), 0)
    q2 = jnp.sum(xq * xq, axis=1, keepdims=True)
    xqb = xq.astype(jnp.bfloat16).astype(jnp.float32)
    qc = [xqb[:, c:c + 1] for c in range(D)]

    lane = jax.lax.broadcasted_iota(jnp.int32, (QB, LANES), 1)
    inf = jnp.full((QB, LANES), jnp.inf, dtype=jnp.float32)

    lv = [inf for _ in range(R)]
    li = [jnp.zeros((QB, LANES), jnp.int32) for _ in range(R)]
    for g in range(G):
        base = g * LANES
        kt = [k0_ref[g:g + 1, :], k1_ref[g:g + 1, :], k2_ref[g:g + 1, :]]
        k2 = kt[0] * kt[0] + kt[1] * kt[1] + kt[2] * kt[2]
        kb = [k.astype(jnp.bfloat16).astype(jnp.float32) for k in kt]
        dot = qc[0] * kb[0] + qc[1] * kb[1] + qc[2] * kb[2]
        d2 = (q2 + k2) - 2.0 * dot
        ki = base + lane
        valid = (ki != rows) & (ki < N)
        v = jnp.where(valid, d2, inf)
        vi = ki
        for r in range(R):
            cmp = v < lv[r]
            lv[r], v = jnp.where(cmp, v, lv[r]), jnp.where(cmp, lv[r], v)
            li[r], vi = jnp.where(cmp, vi, li[r]), jnp.where(cmp, li[r], vi)

    pv = jnp.concatenate(lv, axis=1)
    pi = jnp.concatenate(li, axis=1)

    mprev = jnp.full((QB, 1), -jnp.inf, dtype=jnp.float32)
    iprev = jnp.full((QB, 1), -1, dtype=jnp.int32)
    col = jax.lax.broadcasted_iota(jnp.int32, (QB, LANES), 1)
    res = jnp.zeros((QB, LANES), jnp.int32)
    pinf = jnp.full(pv.shape, jnp.inf, dtype=jnp.float32)
    pbig = jnp.full(pv.shape, BIGI, dtype=jnp.int32)
    for j in range(KNN):
        active = (pv > mprev) | ((pv == mprev) & (pi > iprev))
        vals = jnp.where(active, pv, pinf)
        m = jnp.min(vals, axis=1, keepdims=True)
        idx = jnp.min(jnp.where(vals == m, pi, pbig), axis=1, keepdims=True)
        res = jnp.where(col == j, idx, res)
        mprev, iprev = m, idx
    out_ref[...] = res


def kernel(datapoint):
    x = datapoint.astype(jnp.float32)
    xpad = jnp.pad(x, ((0, NPAD - N), (0, 0)))
    kcoord = [xpad[:, c].reshape(G, LANES) for c in range(D)]
    nbr = pl.pallas_call(
        _knn_body,
        grid=(NPAD // QB,),
        in_specs=[
            pl.BlockSpec((QB, D), lambda s: (s, 0)),
            pl.BlockSpec((G, LANES), lambda s: (0, 0)),
            pl.BlockSpec((G, LANES), lambda s: (0, 0)),
            pl.BlockSpec((G, LANES), lambda s: (0, 0)),
        ],
        out_specs=pl.BlockSpec((QB, LANES), lambda s: (s, 0)),
        out_shape=jax.ShapeDtypeStruct((NPAD, LANES), jnp.int32),
    )(xpad, *kcoord)
    src = nbr[:N, :KNN].reshape(-1)
    dst = jnp.repeat(jnp.arange(N), KNN)
    return jnp.stack([src, dst], axis=0).astype(jnp.int64)


# speedup vs baseline: 5.9799x; 1.8266x over previous
import jax
import jax.numpy as jnp
from jax.experimental import pallas as pl

N = 20000
D = 3
KNN = 17
LANES = 128
G = (N + LANES - 1) // LANES
NPAD = G * LANES
QB = 16
R = 6
BIGI = 2**30


def _knn_body(xq_ref, kb0_ref, kb1_ref, kb2_ref, k2_ref, out_ref):
    s = pl.program_id(0)
    xq = xq_ref[...]
    rows = s * QB + jax.lax.broadcasted_iota(jnp.int32, (QB, 1), 0)
    q2 = jnp.sum(xq * xq, axis=1, keepdims=True)
    xqb = xq.astype(jnp.bfloat16).astype(jnp.float32)
    qc = [xqb[:, c:c + 1] for c in range(D)]

    lane = jax.lax.broadcasted_iota(jnp.int32, (QB, LANES), 1)
    inf = jnp.full((QB, LANES), jnp.inf, dtype=jnp.float32)

    lv = [inf for _ in range(R)]
    li = [jnp.zeros((QB, LANES), jnp.int32) for _ in range(R)]
    for g in range(G):
        kb = [kb0_ref[g:g + 1, :], kb1_ref[g:g + 1, :], kb2_ref[g:g + 1, :]]
        dot = qc[0] * kb[0] + qc[1] * kb[1] + qc[2] * kb[2]
        d2 = (q2 + k2_ref[g:g + 1, :]) - (dot + dot)
        v = d2
        vi = g * LANES + lane
        for r in range(R):
            cmp = v < lv[r]
            lv[r], v = jnp.where(cmp, v, lv[r]), jnp.where(cmp, lv[r], v)
            li[r], vi = jnp.where(cmp, vi, li[r]), jnp.where(cmp, li[r], vi)

    pv = jnp.concatenate(lv, axis=1)
    pi = jnp.concatenate(li, axis=1)
    pv = jnp.where(pi == rows, jnp.inf, pv)

    mprev = jnp.full((QB, 1), -jnp.inf, dtype=jnp.float32)
    iprev = jnp.full((QB, 1), -1, dtype=jnp.int32)
    col = jax.lax.broadcasted_iota(jnp.int32, (QB, LANES), 1)
    res = jnp.zeros((QB, LANES), jnp.int32)
    pinf = jnp.full(pv.shape, jnp.inf, dtype=jnp.float32)
    pbig = jnp.full(pv.shape, BIGI, dtype=jnp.int32)
    for j in range(KNN):
        active = (pv > mprev) | ((pv == mprev) & (pi > iprev))
        vals = jnp.where(active, pv, pinf)
        m = jnp.min(vals, axis=1, keepdims=True)
        idx = jnp.min(jnp.where(vals == m, pi, pbig), axis=1, keepdims=True)
        res = jnp.where(col == j, idx, res)
        mprev, iprev = m, idx
    out_ref[...] = res


def kernel(datapoint):
    x = datapoint.astype(jnp.float32)
    xpad = jnp.pad(x, ((0, NPAD - N), (0, 0)))
    xb = xpad.astype(jnp.bfloat16).astype(jnp.float32)
    kb = [xb[:, c].reshape(G, LANES) for c in range(D)]
    k2 = jnp.sum(xpad * xpad, axis=1)
    k2 = jnp.where(jnp.arange(NPAD) >= N, jnp.inf, k2).reshape(G, LANES)
    nbr = pl.pallas_call(
        _knn_body,
        grid=(NPAD // QB,),
        in_specs=[
            pl.BlockSpec((QB, D), lambda s: (s, 0)),
            pl.BlockSpec((G, LANES), lambda s: (0, 0)),
            pl.BlockSpec((G, LANES), lambda s: (0, 0)),
            pl.BlockSpec((G, LANES), lambda s: (0, 0)),
            pl.BlockSpec((G, LANES), lambda s: (0, 0)),
        ],
        out_specs=pl.BlockSpec((QB, LANES), lambda s: (s, 0)),
        out_shape=jax.ShapeDtypeStruct((NPAD, LANES), jnp.int32),
    )(xpad, *kb, k2)
    src = nbr[:N, :KNN].reshape(-1)
    dst = jnp.repeat(jnp.arange(N), KNN)
    return jnp.stack([src, dst], axis=0).astype(jnp.int64)


# speedup vs baseline: 5.9849x; 1.0008x over previous
import jax
import jax.numpy as jnp
from jax.experimental import pallas as pl

N = 20000
D = 3
KNN = 17
LANES = 128
G = (N + LANES - 1) // LANES
NPAD = G * LANES
QB = 16
R = 6
BIGI = 2**30


def _knn_body(xq_ref, kb0_ref, kb1_ref, kb2_ref, k2_ref, out_ref):
    s = pl.program_id(0)
    xq = xq_ref[...]
    rows = s * QB + jax.lax.broadcasted_iota(jnp.int32, (QB, 1), 0)
    q2 = jnp.sum(xq * xq, axis=1, keepdims=True)
    xqb = xq.astype(jnp.bfloat16).astype(jnp.float32)
    qc = [xqb[:, c:c + 1] for c in range(D)]

    lane = jax.lax.broadcasted_iota(jnp.int32, (QB, LANES), 1)
    inf = jnp.full((QB, LANES), jnp.inf, dtype=jnp.float32)

    lv = [inf for _ in range(R)]
    li = [jnp.zeros((QB, LANES), jnp.int32) for _ in range(R)]
    for g in range(G):
        kb = [kb0_ref[g:g + 1, :], kb1_ref[g:g + 1, :], kb2_ref[g:g + 1, :]]
        dot = qc[0] * kb[0] + qc[1] * kb[1] + qc[2] * kb[2]
        d2 = (q2 + k2_ref[g:g + 1, :]) - (dot + dot)
        v = d2
        vi = g * LANES + lane
        for r in range(R):
            cmp = v < lv[r]
            lv[r], v = jnp.where(cmp, v, lv[r]), jnp.where(cmp, lv[r], v)
            li[r], vi = jnp.where(cmp, vi, li[r]), jnp.where(cmp, li[r], vi)

    pv = jnp.concatenate(lv, axis=1)
    pi = jnp.concatenate(li, axis=1)
    pv = jnp.where(pi == rows, jnp.inf, pv)

    mprev = jnp.full((QB, 1), -jnp.inf, dtype=jnp.float32)
    iprev = jnp.full((QB, 1), -1, dtype=jnp.int32)
    col = jax.lax.broadcasted_iota(jnp.int32, (QB, LANES), 1)
    res = jnp.zeros((QB, LANES), jnp.int32)
    pinf = jnp.full(pv.shape, jnp.inf, dtype=jnp.float32)
    pbig = jnp.full(pv.shape, BIGI, dtype=jnp.int32)
    for j in range(KNN):
        active = (pv > mprev) | ((pv == mprev) & (pi > iprev))
        vals = jnp.where(active, pv, pinf)
        m = jnp.min(vals, axis=1, keepdims=True)
        idx = jnp.min(jnp.where(vals == m, pi, pbig), axis=1, keepdims=True)
        res = jnp.where(col == j, idx, res)
        mprev, iprev = m, idx
    out_ref[...] = res


def kernel(datapoint):
    x = datapoint.astype(jnp.float32)
    xpad = jnp.pad(x, ((0, NPAD - N), (0, 0)))
    xb = jax.lax.reduce_precision(xpad, 8, 7)
    kb = [xb[:, c].reshape(G, LANES) for c in range(D)]
    k2 = jnp.sum(xpad * xpad, axis=1)
    k2 = jnp.where(jnp.arange(NPAD) >= N, jnp.inf, k2).reshape(G, LANES)
    nbr = pl.pallas_call(
        _knn_body,
        grid=(NPAD // QB,),
        in_specs=[
            pl.BlockSpec((QB, D), lambda s: (s, 0)),
            pl.BlockSpec((G, LANES), lambda s: (0, 0)),
            pl.BlockSpec((G, LANES), lambda s: (0, 0)),
            pl.BlockSpec((G, LANES), lambda s: (0, 0)),
            pl.BlockSpec((G, LANES), lambda s: (0, 0)),
        ],
        out_specs=pl.BlockSpec((QB, LANES), lambda s: (s, 0)),
        out_shape=jax.ShapeDtypeStruct((NPAD, LANES), jnp.int32),
    )(xpad, *kb, k2)
    src = nbr[:N, :KNN].reshape(-1)
    dst = jnp.repeat(jnp.arange(N), KNN)
    return jnp.stack([src, dst], axis=0).astype(jnp.int64)


# speedup vs baseline: 13.2338x; 2.2112x over previous
import jax
import jax.numpy as jnp
from jax.experimental import pallas as pl

N = 20000
D = 3
KNN = 17
LANES = 128
G = (N + LANES - 1) // LANES
NPAD = G * LANES
QB = 8
R = 6
NSTREAM = 2
POOL = NSTREAM * R * LANES
QE = 128
BIGI = 2**30


def _build_body(xq_ref, kb0_ref, kb1_ref, kb2_ref, k2_ref, pv_ref, pi_ref):
    xq = xq_ref[...]
    q2 = jnp.sum(xq * xq, axis=1, keepdims=True)
    xqb = xq.astype(jnp.bfloat16).astype(jnp.float32)
    qc = [xqb[:, c:c + 1] for c in range(D)]

    lane = jax.lax.broadcasted_iota(jnp.int32, (QB, LANES), 1)
    inf = jnp.full((QB, LANES), jnp.inf, dtype=jnp.float32)

    lv = [[inf for _ in range(R)] for _ in range(NSTREAM)]
    li = [[jnp.zeros((QB, LANES), jnp.int32) for _ in range(R)]
          for _ in range(NSTREAM)]
    for g in range(G):
        kb = [kb0_ref[g:g + 1, :], kb1_ref[g:g + 1, :], kb2_ref[g:g + 1, :]]
        dot = qc[0] * kb[0] + qc[1] * kb[1] + qc[2] * kb[2]
        d2 = (q2 + k2_ref[g:g + 1, :]) - (dot + dot)
        v = d2
        vi = g * LANES + lane
        slv, sli = lv[g % NSTREAM], li[g % NSTREAM]
        for r in range(R):
            cmp = v < slv[r]
            slv[r], v = jnp.where(cmp, v, slv[r]), jnp.where(cmp, slv[r], v)
            sli[r], vi = jnp.where(cmp, vi, sli[r]), jnp.where(cmp, sli[r], vi)

    pv_ref[...] = jnp.concatenate(lv[0] + lv[1], axis=1)
    pi_ref[...] = jnp.concatenate(li[0] + li[1], axis=1)


def _extract_body(pv_ref, pi_ref, out_ref):
    s = pl.program_id(0)
    rows = s * QE + jax.lax.broadcasted_iota(jnp.int32, (QE, 1), 0)
    pv = pv_ref[...]
    pi = pi_ref[...]
    pv = jnp.where(pi == rows, jnp.inf, pv)

    mprev = jnp.full((QE, 1), -jnp.inf, dtype=jnp.float32)
    iprev = jnp.full((QE, 1), -1, dtype=jnp.int32)
    col = jax.lax.broadcasted_iota(jnp.int32, (QE, LANES), 1)
    res = jnp.zeros((QE, LANES), jnp.int32)
    pinf = jnp.full(pv.shape, jnp.inf, dtype=jnp.float32)
    pbig = jnp.full(pv.shape, BIGI, dtype=jnp.int32)
    for j in range(KNN):
        active = (pv > mprev) | ((pv == mprev) & (pi > iprev))
        vals = jnp.where(active, pv, pinf)
        m = jnp.min(vals, axis=1, keepdims=True)
        idx = jnp.min(jnp.where(vals == m, pi, pbig), axis=1, keepdims=True)
        res = jnp.where(col == j, idx, res)
        mprev, iprev = m, idx
    out_ref[...] = res


def kernel(datapoint):
    x = datapoint.astype(jnp.float32)
    xpad = jnp.pad(x, ((0, NPAD - N), (0, 0)))
    xb = jax.lax.reduce_precision(xpad, 8, 7)
    kb = [xb[:, c].reshape(G, LANES) for c in range(D)]
    k2 = jnp.sum(xpad * xpad, axis=1)
    k2 = jnp.where(jnp.arange(NPAD) >= N, jnp.inf, k2).reshape(G, LANES)
    pv, pi = pl.pallas_call(
        _build_body,
        grid=(NPAD // QB,),
        in_specs=[
            pl.BlockSpec((QB, D), lambda s: (s, 0)),
            pl.BlockSpec((G, LANES), lambda s: (0, 0)),
            pl.BlockSpec((G, LANES), lambda s: (0, 0)),
            pl.BlockSpec((G, LANES), lambda s: (0, 0)),
            pl.BlockSpec((G, LANES), lambda s: (0, 0)),
        ],
        out_specs=[
            pl.BlockSpec((QB, POOL), lambda s: (s, 0)),
            pl.BlockSpec((QB, POOL), lambda s: (s, 0)),
        ],
        out_shape=[
            jax.ShapeDtypeStruct((NPAD, POOL), jnp.float32),
            jax.ShapeDtypeStruct((NPAD, POOL), jnp.int32),
        ],
    )(xpad, *kb, k2)
    nbr = pl.pallas_call(
        _extract_body,
        grid=(NPAD // QE,),
        in_specs=[
            pl.BlockSpec((QE, POOL), lambda s: (s, 0)),
            pl.BlockSpec((QE, POOL), lambda s: (s, 0)),
        ],
        out_specs=pl.BlockSpec((QE, LANES), lambda s: (s, 0)),
        out_shape=jax.ShapeDtypeStruct((NPAD, LANES), jnp.int32),
    )(pv, pi)
    src = nbr[:N, :KNN].reshape(-1)
    dst = jnp.repeat(jnp.arange(N), KNN)
    return jnp.stack([src, dst], axis=0).astype(jnp.int64)


# speedup vs baseline: 16.3379x; 1.2346x over previous
import jax
import jax.numpy as jnp
from jax.experimental import pallas as pl

N = 20000
D = 3
KNN = 17
LANES = 128
G = (N + LANES - 1) // LANES
NPAD = G * LANES
QB = 16
R = 5
NSTREAM = 2
POOL = NSTREAM * R * LANES
QE = 128
BIGI = 2**30


def _build_body(xq_ref, kb0_ref, kb1_ref, kb2_ref, k2_ref, pv_ref, pi_ref):
    xq = xq_ref[...]
    q2 = jnp.sum(xq * xq, axis=1, keepdims=True)
    xqb = xq.astype(jnp.bfloat16).astype(jnp.float32)
    qc = [xqb[:, c:c + 1] for c in range(D)]

    lane = jax.lax.broadcasted_iota(jnp.int32, (QB, LANES), 1)
    inf = jnp.full((QB, LANES), jnp.inf, dtype=jnp.float32)

    lv = [[inf for _ in range(R)] for _ in range(NSTREAM)]
    li = [[jnp.zeros((QB, LANES), jnp.int32) for _ in range(R)]
          for _ in range(NSTREAM)]
    for g in range(G):
        kb = [kb0_ref[g:g + 1, :], kb1_ref[g:g + 1, :], kb2_ref[g:g + 1, :]]
        dot = qc[0] * kb[0] + qc[1] * kb[1] + qc[2] * kb[2]
        d2 = (q2 + k2_ref[g:g + 1, :]) - (dot + dot)
        v = d2
        vi = g * LANES + lane
        slv, sli = lv[g % NSTREAM], li[g % NSTREAM]
        for r in range(R):
            cmp = v < slv[r]
            slv[r], v = jnp.where(cmp, v, slv[r]), jnp.where(cmp, slv[r], v)
            sli[r], vi = jnp.where(cmp, vi, sli[r]), jnp.where(cmp, sli[r], vi)

    pv_ref[...] = jnp.concatenate(lv[0] + lv[1], axis=1)
    pi_ref[...] = jnp.concatenate(li[0] + li[1], axis=1)


def _extract_body(pv_ref, pi_ref, out_ref):
    s = pl.program_id(0)
    rows = s * QE + jax.lax.broadcasted_iota(jnp.int32, (QE, 1), 0)
    pv = pv_ref[...]
    pi = pi_ref[...]
    pv = jnp.where(pi == rows, jnp.inf, pv)

    col = jax.lax.broadcasted_iota(jnp.int32, (QE, LANES), 1)
    res = jnp.zeros((QE, LANES), jnp.int32)
    pinf = jnp.full(pv.shape, jnp.inf, dtype=jnp.float32)
    pbig = jnp.full(pv.shape, BIGI, dtype=jnp.int32)
    for j in range(KNN):
        m = jnp.min(pv, axis=1, keepdims=True)
        veq = pv == m
        idx = jnp.min(jnp.where(veq, pi, pbig), axis=1, keepdims=True)
        pv = jnp.where(veq & (pi == idx), pinf, pv)
        res = jnp.where(col == j, idx, res)
    out_ref[...] = res


def kernel(datapoint):
    x = datapoint.astype(jnp.float32)
    xpad = jnp.pad(x, ((0, NPAD - N), (0, 0)))
    xb = jax.lax.reduce_precision(xpad, 8, 7)
    kb = [xb[:, c].reshape(G, LANES) for c in range(D)]
    k2 = jnp.sum(xpad * xpad, axis=1)
    k2 = jnp.where(jnp.arange(NPAD) >= N, jnp.inf, k2).reshape(G, LANES)
    pv, pi = pl.pallas_call(
        _build_body,
        grid=(NPAD // QB,),
        in_specs=[
            pl.BlockSpec((QB, D), lambda s: (s, 0)),
            pl.BlockSpec((G, LANES), lambda s: (0, 0)),
            pl.BlockSpec((G, LANES), lambda s: (0, 0)),
            pl.BlockSpec((G, LANES), lambda s: (0, 0)),
            pl.BlockSpec((G, LANES), lambda s: (0, 0)),
        ],
        out_specs=[
            pl.BlockSpec((QB, POOL), lambda s: (s, 0)),
            pl.BlockSpec((QB, POOL), lambda s: (s, 0)),
        ],
        out_shape=[
            jax.ShapeDtypeStruct((NPAD, POOL), jnp.float32),
            jax.ShapeDtypeStruct((NPAD, POOL), jnp.int32),
        ],
    )(xpad, *kb, k2)
    nbr = pl.pallas_call(
        _extract_body,
        grid=(NPAD // QE,),
        in_specs=[
            pl.BlockSpec((QE, POOL), lambda s: (s, 0)),
            pl.BlockSpec((QE, POOL), lambda s: (s, 0)),
        ],
        out_specs=pl.BlockSpec((QE, LANES), lambda s: (s, 0)),
        out_shape=jax.ShapeDtypeStruct((NPAD, LANES), jnp.int32),
    )(pv, pi)
    src = nbr[:N, :KNN].reshape(-1)
    dst = jnp.repeat(jnp.arange(N), KNN)
    return jnp.stack([src, dst], axis=0).astype(jnp.int64)


# speedup vs baseline: 18.1151x; 1.1088x over previous
import jax
import jax.numpy as jnp
from jax.experimental import pallas as pl

N = 20000
D = 3
KNN = 17
LANES = 128
G = (N + LANES - 1) // LANES
NPAD = G * LANES
QB = 16
R = 5
NSTREAM = 2
POOL = NSTREAM * R * LANES
QE = 128
BIGI = 2**30


def _build_body(xq_ref, xqb_ref, kbt_ref, k2_ref, pv_ref, pi_ref):
    xq = xq_ref[...]
    q2 = jnp.sum(xq * xq, axis=1, keepdims=True)
    dot = jax.lax.dot_general(
        xqb_ref[...], kbt_ref[...], (((1,), (0,)), ((), ())),
        preferred_element_type=jnp.float32)

    lane = jax.lax.broadcasted_iota(jnp.int32, (QB, LANES), 1)
    inf = jnp.full((QB, LANES), jnp.inf, dtype=jnp.float32)

    lv = [[inf for _ in range(R)] for _ in range(NSTREAM)]
    li = [[jnp.zeros((QB, LANES), jnp.int32) for _ in range(R)]
          for _ in range(NSTREAM)]
    for g in range(G):
        ds = dot[:, g * LANES:(g + 1) * LANES]
        d2 = (q2 + k2_ref[g:g + 1, :]) - (ds + ds)
        v = d2
        vi = g * LANES + lane
        slv, sli = lv[g % NSTREAM], li[g % NSTREAM]
        for r in range(R - 1):
            cmp = v < slv[r]
            slv[r], v = jnp.where(cmp, v, slv[r]), jnp.where(cmp, slv[r], v)
            sli[r], vi = jnp.where(cmp, vi, sli[r]), jnp.where(cmp, sli[r], vi)
        cmp = v < slv[R - 1]
        slv[R - 1] = jnp.where(cmp, v, slv[R - 1])
        sli[R - 1] = jnp.where(cmp, vi, sli[R - 1])

    pv_ref[...] = jnp.concatenate(lv[0] + lv[1], axis=1)
    pi_ref[...] = jnp.concatenate(li[0] + li[1], axis=1)


def _extract_body(pv_ref, pi_ref, out_ref):
    s = pl.program_id(0)
    rows = s * QE + jax.lax.broadcasted_iota(jnp.int32, (QE, 1), 0)
    pv = pv_ref[...]
    pi = pi_ref[...]
    pv = jnp.where(pi == rows, jnp.inf, pv)

    col = jax.lax.broadcasted_iota(jnp.int32, (QE, LANES), 1)
    res = jnp.zeros((QE, LANES), jnp.int32)
    pinf = jnp.full(pv.shape, jnp.inf, dtype=jnp.float32)
    pbig = jnp.full(pv.shape, BIGI, dtype=jnp.int32)
    for j in range(KNN):
        m = jnp.min(pv, axis=1, keepdims=True)
        veq = pv == m
        idx = jnp.min(jnp.where(veq, pi, pbig), axis=1, keepdims=True)
        pv = jnp.where(veq & (pi == idx), pinf, pv)
        res = jnp.where(col == j, idx, res)
    out_ref[...] = res


def kernel(datapoint):
    x = datapoint.astype(jnp.float32)
    xpad = jnp.pad(x, ((0, NPAD - N), (0, 0)))
    xbf = xpad.astype(jnp.bfloat16)
    kbt = xbf.T
    k2 = jnp.sum(xpad * xpad, axis=1)
    k2 = jnp.where(jnp.arange(NPAD) >= N, jnp.inf, k2).reshape(G, LANES)
    pv, pi = pl.pallas_call(
        _build_body,
        grid=(NPAD // QB,),
        in_specs=[
            pl.BlockSpec((QB, D), lambda s: (s, 0)),
            pl.BlockSpec((QB, D), lambda s: (s, 0)),
            pl.BlockSpec((D, NPAD), lambda s: (0, 0)),
            pl.BlockSpec((G, LANES), lambda s: (0, 0)),
        ],
        out_specs=[
            pl.BlockSpec((QB, POOL), lambda s: (s, 0)),
            pl.BlockSpec((QB, POOL), lambda s: (s, 0)),
        ],
        out_shape=[
            jax.ShapeDtypeStruct((NPAD, POOL), jnp.float32),
            jax.ShapeDtypeStruct((NPAD, POOL), jnp.int32),
        ],
    )(xpad, xbf, kbt, k2)
    nbr = pl.pallas_call(
        _extract_body,
        grid=(NPAD // QE,),
        in_specs=[
            pl.BlockSpec((QE, POOL), lambda s: (s, 0)),
            pl.BlockSpec((QE, POOL), lambda s: (s, 0)),
        ],
        out_specs=pl.BlockSpec((QE, LANES), lambda s: (s, 0)),
        out_shape=jax.ShapeDtypeStruct((NPAD, LANES), jnp.int32),
    )(pv, pi)
    src = nbr[:N, :KNN].reshape(-1)
    dst = jnp.repeat(jnp.arange(N), KNN)
    return jnp.stack([src, dst], axis=0).astype(jnp.int64)
